# baseline (speedup 1.0000x reference)
import functools

import jax
import jax.numpy as jnp
import numpy as np
from jax import lax
from jax.experimental import pallas as pl
from jax.experimental.pallas import tpu as pltpu
from jax.experimental.pallas import tpu_sc as plsc

B, T = 4096, 12
N = B * T
D = 256
NUM_STATUS = 7

_NC, _NS = 2, 16
_NW = _NC * _NS
_KSL = 4
_NSL = N // _KSL
_ROWS_PER_TILE = _NSL // _NW
_SLAB = 128
_E = 32
_SUB = _SLAB // _E
_SLABS = _ROWS_PER_TILE // _SLAB


def _hp_multihot_np(num_tokens=1024, n_bins=64):
    values = np.arange(num_tokens)
    arr = (np.arange(n_bins)[None] < np.floor(n_bins * values / num_tokens)[:, None]).astype(float)
    extra = values % (num_tokens / n_bins) / (num_tokens / n_bins)
    extra = 2 * extra - 1
    extra_mask = (np.arange(n_bins)[None] <= np.floor(n_bins * values / num_tokens)[:, None]) - arr
    arr = arr + extra_mask * extra[:, None]
    arr = np.where(arr <= 0, -1.0, arr)
    return np.asarray(arr, dtype=np.float32)


_HP_TAB = _hp_multihot_np()


def _ln(x):
    mu = jnp.mean(x, axis=-1, keepdims=True)
    var = jnp.var(x, axis=-1, keepdims=True)
    return (x - mu) * lax.rsqrt(var + 1e-5)


def _dot(a, b):
    return jnp.dot(a, b, preferred_element_type=jnp.float32)


def _tables_body(sp_tab, sp_lw, sp_lb, it_tab, it_lw, it_lb, ab_tab, ab_lw,
                 ab_lb, mv_tab, mv_lw, mv_lb, rest_w, rest_b, hp_tab,
                 mw_sp, mw_ab, mw_it,
                 o_sp, o_ab, o_it, o_mv, o_lm, o_hp, o_168, o_16):
    o_sp[...] = _dot(_ln(_dot(sp_tab[...], sp_lw[...]) + sp_lb[...]), mw_sp[...])
    o_ab[...] = _dot(_ln(_dot(ab_tab[...], ab_lw[...]) + ab_lb[...]), mw_ab[...])
    o_it[...] = _dot(_ln(_dot(it_tab[...], it_lw[...]) + it_lb[...]), mw_it[...])
    tmv = _dot(mv_tab[...], mv_lw[...]) + mv_lb[...]
    o_mv[...] = tmv
    w = rest_w[...]
    o_lm[...] = _dot(tmv, w[0:256])
    o_hp[...] = _dot(hp_tab[...], w[256:320])

    def onehot(rows, vals, div, mod):
        r = lax.broadcasted_iota(jnp.int32, (rows, vals), 0)
        c = lax.broadcasted_iota(jnp.int32, (rows, vals), 1)
        return jnp.where((r // div) % mod == c, 1.0, 0.0).astype(jnp.float32)

    o_168[...] = (_dot(onehot(168, NUM_STATUS, 24, NUM_STATUS), w[324:331]) +
                  _dot(onehot(168, 4, 6, 4), w[335:339]) +
                  _dot(onehot(168, 6, 1, 6), w[339:345]))
    o_16[...] = (_dot(onehot(16, 2, 8, 2), w[320:322]) +
                 _dot(onehot(16, 2, 4, 2), w[322:324]) +
                 _dot(onehot(16, 2, 2, 2), w[333:335]) +
                 _dot(onehot(16, 2, 1, 2), w[331:333]) + rest_b[...])


def _build_tables(sp_tab, sp_lw, sp_lb, it_tab, it_lw, it_lb, ab_tab, ab_lw,
                  ab_lb, mv_tab, mv_lw, mv_lb, rest_w, rest_b, hp_tab,
                  mw_sp, mw_ab, mw_it):
    outs = [jax.ShapeDtypeStruct((r, D), jnp.float32)
            for r in (450, 200, 300, 400, 400, 1024, 168, 16)]
    return pl.pallas_call(_tables_body, out_shape=outs)(
        sp_tab, sp_lw, sp_lb, it_tab, it_lw, it_lb, ab_tab, ab_lw, ab_lb,
        mv_tab, mv_lw, mv_lb, rest_w, rest_b, hp_tab, mw_sp, mw_ab, mw_it)


_EC = 24


def _sc_body(koff, ent, mv1d, tsp, tab, tit, tmv, tlm, thp, t168, t16,
             out3, ent_v, cidx, mvi, a0, a1, a2, mvb, r0, r1, r2, r3, sumb,
             s0, s1):
    wid = lax.axis_index("s") * _NC + lax.axis_index("c")

    def slab(si, carry):
        sbase = wid * _ROWS_PER_TILE + si * _SLAB
        pltpu.sync_copy(ent.at[:, pl.ds(koff + sbase, _SLAB)], ent_v)
        pltpu.sync_copy(mv1d.at[pl.ds((koff + sbase) * 4, _SLAB * 4)], mvi)
        for i in range(_SLAB // 16):
            sl = pl.ds(i * 16, 16)
            st = ent_v[6, sl]
            slp = jnp.minimum(jnp.maximum(ent_v[10, sl], 0), 3)
            tx = jnp.minimum(jnp.maximum(ent_v[11, sl], 0), 5)
            cidx[0, sl] = st * 24 + slp * 6 + tx
            cidx[1, sl] = (ent_v[4, sl] * 8 + ent_v[5, sl] * 4 +
                           ent_v[8, sl] * 2 + ent_v[9, sl])

        def chunk(sub, carry2):
            base = sbase + sub * _E
            ix = lambda row: ent_v.at[row, pl.ds(sub * _E, _E)]
            gathers = [
                pltpu.async_copy(tsp.at[ix(0)], a0, s0),
                pltpu.async_copy(tab.at[ix(2)], a1, s0),
                pltpu.async_copy(tit.at[ix(1)], a2, s0),
                pltpu.async_copy(tmv.at[mvi.at[pl.ds(sub * 4 * _E, 4 * _E)]],
                                 mvb, s0),
                pltpu.async_copy(tlm.at[ix(7)], r0, s0),
                pltpu.async_copy(thp.at[ix(3)], r1, s0),
                pltpu.async_copy(t168.at[cidx.at[0, pl.ds(sub * _E, _E)]], r2, s0),
                pltpu.async_copy(t16.at[cidx.at[1, pl.ds(sub * _E, _E)]], r3, s0),
            ]
            for c in gathers:
                c.wait()

            @pl.when(si * _SUB + sub > 0)
            def _drain():
                pltpu.make_async_copy(sumb, out3.at[pl.ds(base, _E)], s1).wait()

            def sum_grp(g, carry3):
                o0 = pl.ds(pl.multiple_of(g * 32, 32), 32)
                o1 = pl.ds(pl.multiple_of(g * 32 + D, 32), 32)
                o2 = pl.ds(pl.multiple_of(g * 32 + 2 * D, 32), 32)
                for r in range(_E):
                    sumb[r, o0] = a0[r, o0] + a1[r, o0] + a2[r, o0]
                    sumb[r, o1] = (mvb[r, o0] + mvb[r + _E, o0] +
                                   mvb[r + 2 * _E, o0] + mvb[r + 3 * _E, o0])
                    sumb[r, o2] = (r0[r, o0] + r1[r, o0] +
                                   r2[r, o0] + r3[r, o0])
                return carry3

            lax.fori_loop(0, D // 32, sum_grp, 0)
            pltpu.async_copy(sumb, out3.at[pl.ds(base, _E)], s1)
            return carry2

        return lax.fori_loop(0, _SUB, chunk, carry)

    lax.fori_loop(0, _SLABS, slab, 0)
    pltpu.make_async_copy(sumb, out3.at[pl.ds(0, _E)], s1).wait()


def _sc_gather(koff, ent24, mv1d, tsp, tab, tit, tmv, tlm, thp, t168, t16):
    mesh = plsc.VectorSubcoreMesh(core_axis_name="c", subcore_axis_name="s")

    def entry(ent, mv1d, tsp, tab, tit, tmv, tlm, thp, t168, t16, out3,
              ent_v, cidx, mvi, a0, a1, a2, mvb, r0, r1, r2, r3, sumb,
              s0, s1):
        _sc_body(koff, ent, mv1d, tsp, tab, tit, tmv, tlm, thp, t168, t16,
                 out3, ent_v, cidx, mvi, a0, a1, a2, mvb, r0, r1, r2, r3,
                 sumb, s0, s1)

    f = pl.kernel(
        entry, mesh=mesh,
        out_type=jax.ShapeDtypeStruct((_NSL, 3 * D), jnp.float32),
        scratch_types=(
            [pltpu.VMEM((_EC, _SLAB), jnp.int32),
             pltpu.VMEM((2, _SLAB), jnp.int32),
             pltpu.VMEM((_SLAB * 4,), jnp.int32)] +
            [pltpu.VMEM((_E, D), jnp.float32)] * 3 +
            [pltpu.VMEM((4 * _E, D), jnp.float32)] +
            [pltpu.VMEM((_E, D), jnp.float32)] * 4 +
            [pltpu.VMEM((_E, 3 * D), jnp.float32)] +
            [pltpu.SemaphoreType.DMA, pltpu.SemaphoreType.DMA]
        ))
    return f(ent24, mv1d, tsp, tab, tit, tmv, tlm, thp, t168, t16)


def _transpose_body(x, out):
    out[...] = x[...].T


def _transpose_ent(ents):
    return pl.pallas_call(
        _transpose_body,
        grid=(N // 2048,),
        in_specs=[pl.BlockSpec((2048, _EC), lambda i: (i, 0))],
        out_specs=pl.BlockSpec((_EC, 2048), lambda i: (0, i)),
        out_shape=jax.ShapeDtypeStruct((_EC, N), jnp.int32),
    )(ents)


_RB = 1024


def _finish_body(prev, acc, ms, rest, wm, wr, b_sp, b_ab, b_it, b_ms, b_rs,
                 out):
    del prev
    bias = b_sp[...] + b_ab[...] + b_it[...] + b_ms[...] + b_rs[...]
    b16 = lambda x: x.astype(jnp.bfloat16)
    out[...] = (acc[...] + bias +
                _dot(b16(_ln(ms[...] * 0.25)), b16(wm[...])) +
                _dot(b16(_ln(rest[...])), b16(wr[...])))


def _finish(out_prev, k, part3, wm, wr, b_sp, b_ab, b_it, b_ms, b_rs):
    blk = _NSL // _RB
    col = lambda j: pl.BlockSpec((_RB, D), lambda i, j=j: (i, j))
    row_out = pl.BlockSpec((_RB, D), lambda i: (k * blk + i, 0))
    full = pl.BlockSpec((D, D), lambda i: (0, 0))
    vec = pl.BlockSpec((1, D), lambda i: (0, 0))
    return pl.pallas_call(
        _finish_body,
        grid=(blk,),
        in_specs=[row_out, col(0), col(1), col(2), full, full,
                  vec, vec, vec, vec, vec],
        out_specs=row_out,
        out_shape=jax.ShapeDtypeStruct((N, D), jnp.float32),
        input_output_aliases={0: 0},
    )(out_prev, part3, part3, part3, wm, wr, b_sp, b_ab, b_it, b_ms, b_rs)


def kernel(entities, species_tab, species_lw, species_lb, item_tab, item_lw,
           item_lb, ability_tab, ability_lw, ability_lb, moves_tab, moves_lw,
           moves_lb, rest_W, rest_b, mW_species, mb_species, mW_ability,
           mb_ability, mW_item, mb_item, mW_moveset, mb_moveset, mW_rest,
           mb_rest):
    r1 = lambda v: v.reshape(1, D)
    tsp, tab, tit, tmv, tlm, thp, t168, t16 = _build_tables(
        species_tab, species_lw, r1(species_lb), item_tab, item_lw,
        r1(item_lb), ability_tab, ability_lw, r1(ability_lb), moves_tab,
        moves_lw, r1(moves_lb), rest_W, r1(rest_b), jnp.asarray(_HP_TAB),
        mW_species, mW_ability, mW_item)

    ents2 = jnp.asarray(entities, jnp.int32).reshape(N, _EC)
    ent24 = _transpose_ent(ents2)
    mv1d = ents2[:, 20:24].reshape(N // _E, _E, 4).transpose(0, 2, 1).reshape(-1)

    parts = [_sc_gather(k * _NSL, ent24, mv1d, tsp, tab, tit, tmv, tlm, thp,
                        t168, t16) for k in range(_KSL)]
    out = jnp.zeros((N, D), jnp.float32)
    for k, part3 in enumerate(parts):
        out = _finish(out, k, part3, mW_moveset, mW_rest,
                      r1(mb_species), r1(mb_ability), r1(mb_item),
                      r1(mb_moveset), r1(mb_rest))
    return out.reshape(B, T, D)


# speedup vs baseline: 1.2066x; 1.2066x over previous
import functools

import jax
import jax.numpy as jnp
import numpy as np
from jax import lax
from jax.experimental import pallas as pl
from jax.experimental.pallas import tpu as pltpu
from jax.experimental.pallas import tpu_sc as plsc

B, T = 4096, 12
N = B * T
D = 256
NUM_STATUS = 7

_NC, _NS = 2, 16
_NW = _NC * _NS
_KSL = 4
_NSL = N // _KSL
_ROWS_PER_TILE = _NSL // _NW
_SLAB = 128
_E = 32
_SUB = _SLAB // _E
_SLABS = _ROWS_PER_TILE // _SLAB


def _hp_multihot_np(num_tokens=1024, n_bins=64):
    values = np.arange(num_tokens)
    arr = (np.arange(n_bins)[None] < np.floor(n_bins * values / num_tokens)[:, None]).astype(float)
    extra = values % (num_tokens / n_bins) / (num_tokens / n_bins)
    extra = 2 * extra - 1
    extra_mask = (np.arange(n_bins)[None] <= np.floor(n_bins * values / num_tokens)[:, None]) - arr
    arr = arr + extra_mask * extra[:, None]
    arr = np.where(arr <= 0, -1.0, arr)
    return np.asarray(arr, dtype=np.float32)


_HP_TAB = _hp_multihot_np()


def _ln(x):
    mu = jnp.mean(x, axis=-1, keepdims=True)
    var = jnp.var(x, axis=-1, keepdims=True)
    return (x - mu) * lax.rsqrt(var + 1e-5)


def _dot(a, b):
    return jnp.dot(a, b, preferred_element_type=jnp.float32)


def _tables_body(sp_tab, sp_lw, sp_lb, it_tab, it_lw, it_lb, ab_tab, ab_lw,
                 ab_lb, mv_tab, mv_lw, mv_lb, rest_w, rest_b, hp_tab,
                 mw_sp, mw_ab, mw_it,
                 o_sp, o_ab, o_it, o_mv, o_lm, o_hp, o_168, o_16):
    o_sp[...] = _dot(_ln(_dot(sp_tab[...], sp_lw[...]) + sp_lb[...]), mw_sp[...])
    o_ab[...] = _dot(_ln(_dot(ab_tab[...], ab_lw[...]) + ab_lb[...]), mw_ab[...])
    o_it[...] = _dot(_ln(_dot(it_tab[...], it_lw[...]) + it_lb[...]), mw_it[...])
    tmv = _dot(mv_tab[...], mv_lw[...]) + mv_lb[...]
    o_mv[...] = tmv
    w = rest_w[...]
    o_lm[...] = _dot(tmv, w[0:256])
    o_hp[...] = _dot(hp_tab[...], w[256:320])

    def onehot(rows, vals, div, mod):
        r = lax.broadcasted_iota(jnp.int32, (rows, vals), 0)
        c = lax.broadcasted_iota(jnp.int32, (rows, vals), 1)
        return jnp.where((r // div) % mod == c, 1.0, 0.0).astype(jnp.float32)

    o_168[...] = (_dot(onehot(168, NUM_STATUS, 24, NUM_STATUS), w[324:331]) +
                  _dot(onehot(168, 4, 6, 4), w[335:339]) +
                  _dot(onehot(168, 6, 1, 6), w[339:345]))
    o_16[...] = (_dot(onehot(16, 2, 8, 2), w[320:322]) +
                 _dot(onehot(16, 2, 4, 2), w[322:324]) +
                 _dot(onehot(16, 2, 2, 2), w[333:335]) +
                 _dot(onehot(16, 2, 1, 2), w[331:333]) + rest_b[...])


def _build_tables(sp_tab, sp_lw, sp_lb, it_tab, it_lw, it_lb, ab_tab, ab_lw,
                  ab_lb, mv_tab, mv_lw, mv_lb, rest_w, rest_b, hp_tab,
                  mw_sp, mw_ab, mw_it):
    outs = [jax.ShapeDtypeStruct((r, D), jnp.float32)
            for r in (450, 200, 300, 400, 400, 1024, 168, 16)]
    return pl.pallas_call(_tables_body, out_shape=outs)(
        sp_tab, sp_lw, sp_lb, it_tab, it_lw, it_lb, ab_tab, ab_lw, ab_lb,
        mv_tab, mv_lw, mv_lb, rest_w, rest_b, hp_tab, mw_sp, mw_ab, mw_it)


_EC = 24


def _sc_body(koff, ent, tsp, tab, tit, tmv, tlm, thp, t168, t16,
             acc_out, ms_out, rest_out,
             ent_v, cidx, bufs, s0, s1):
    wid = lax.axis_index("s") * _NC + lax.axis_index("c")

    def slab(si, carry):
        sbase = wid * _ROWS_PER_TILE + si * _SLAB
        pltpu.sync_copy(ent.at[:, pl.ds(koff + sbase, _SLAB)], ent_v)
        for i in range(_SLAB // 16):
            sl = pl.ds(i * 16, 16)
            st = ent_v[6, sl]
            slp = jnp.minimum(jnp.maximum(ent_v[10, sl], 0), 3)
            tx = jnp.minimum(jnp.maximum(ent_v[11, sl], 0), 5)
            cidx[0, sl] = st * 24 + slp * 6 + tx
            cidx[1, sl] = (ent_v[4, sl] * 8 + ent_v[5, sl] * 4 +
                           ent_v[8, sl] * 2 + ent_v[9, sl])

        def chunk(sub, carry2):
            base = sbase + sub * _E
            ix = lambda row: ent_v.at[row, pl.ds(sub * _E, _E)]
            gathers = [
                pltpu.async_copy(tsp.at[ix(0)], bufs[0], s0),
                pltpu.async_copy(tab.at[ix(2)], bufs[1], s0),
                pltpu.async_copy(tit.at[ix(1)], bufs[2], s0),
                pltpu.async_copy(tmv.at[ix(20)], bufs[3], s0),
                pltpu.async_copy(tmv.at[ix(21)], bufs[4], s0),
                pltpu.async_copy(tmv.at[ix(22)], bufs[5], s0),
                pltpu.async_copy(tmv.at[ix(23)], bufs[6], s0),
                pltpu.async_copy(tlm.at[ix(7)], bufs[7], s0),
                pltpu.async_copy(thp.at[ix(3)], bufs[8], s0),
                pltpu.async_copy(t168.at[cidx.at[0, pl.ds(sub * _E, _E)]], bufs[9], s0),
                pltpu.async_copy(t16.at[cidx.at[1, pl.ds(sub * _E, _E)]], bufs[10], s0),
            ]
            for c in gathers:
                c.wait()

            @pl.when(si * _SUB + sub > 0)
            def _drain():
                for b in (bufs[11], bufs[12], bufs[13]):
                    pltpu.make_async_copy(b, acc_out.at[pl.ds(base, _E)], s1).wait()

            def sum_grp(g, carry3):
                sl = pl.ds(pl.multiple_of(g * 32, 32), 32)
                for r in range(_E):
                    bufs[11][r, sl] = bufs[0][r, sl] + bufs[1][r, sl] + bufs[2][r, sl]
                    bufs[12][r, sl] = (bufs[3][r, sl] + bufs[4][r, sl] +
                                       bufs[5][r, sl] + bufs[6][r, sl])
                    bufs[13][r, sl] = (bufs[7][r, sl] + bufs[8][r, sl] +
                                       bufs[9][r, sl] + bufs[10][r, sl])
                return carry3

            lax.fori_loop(0, D // 32, sum_grp, 0)
            pltpu.async_copy(bufs[11], acc_out.at[pl.ds(base, _E)], s1)
            pltpu.async_copy(bufs[12], ms_out.at[pl.ds(base, _E)], s1)
            pltpu.async_copy(bufs[13], rest_out.at[pl.ds(base, _E)], s1)
            return carry2

        return lax.fori_loop(0, _SUB, chunk, carry)

    lax.fori_loop(0, _SLABS, slab, 0)
    for b in (bufs[11], bufs[12], bufs[13]):
        pltpu.make_async_copy(b, acc_out.at[pl.ds(0, _E)], s1).wait()


def _sc_gather(koff, ent24, tsp, tab, tit, tmv, tlm, thp, t168, t16):
    mesh = plsc.VectorSubcoreMesh(core_axis_name="c", subcore_axis_name="s")

    def entry(ent, tsp, tab, tit, tmv, tlm, thp, t168, t16,
              acc_out, ms_out, rest_out, ent_v, cidx,
              b0, b1, b2, b3, b4, b5, b6, b7, b8, b9, b10, b11, b12, b13,
              s0, s1):
        _sc_body(koff, ent, tsp, tab, tit, tmv, tlm, thp, t168, t16,
                 acc_out, ms_out, rest_out, ent_v, cidx,
                 [b0, b1, b2, b3, b4, b5, b6, b7, b8, b9, b10, b11, b12, b13],
                 s0, s1)

    f = pl.kernel(
        entry, mesh=mesh,
        out_type=[jax.ShapeDtypeStruct((_NSL, D), jnp.float32)] * 3,
        scratch_types=(
            [pltpu.VMEM((_EC, _SLAB), jnp.int32),
             pltpu.VMEM((2, _SLAB), jnp.int32)] +
            [pltpu.VMEM((_E, D), jnp.float32)] * 14 +
            [pltpu.SemaphoreType.DMA, pltpu.SemaphoreType.DMA]
        ))
    return f(ent24, tsp, tab, tit, tmv, tlm, thp, t168, t16)


def _transpose_body(x, out):
    out[...] = x[...].T


def _transpose_ent(ents):
    return pl.pallas_call(
        _transpose_body,
        grid=(N // 2048,),
        in_specs=[pl.BlockSpec((2048, _EC), lambda i: (i, 0))],
        out_specs=pl.BlockSpec((_EC, 2048), lambda i: (0, i)),
        out_shape=jax.ShapeDtypeStruct((_EC, N), jnp.int32),
    )(ents)


_RB = 1024


def _finish_body(prev, acc, ms, rest, wm, wr, b_sp, b_ab, b_it, b_ms, b_rs,
                 out):
    del prev
    bias = b_sp[...] + b_ab[...] + b_it[...] + b_ms[...] + b_rs[...]
    b16 = lambda x: x.astype(jnp.bfloat16)
    out[...] = (acc[...] + bias +
                _dot(b16(_ln(ms[...] * 0.25)), b16(wm[...])) +
                _dot(b16(_ln(rest[...])), b16(wr[...])))


def _finish(out_prev, k, acc, ms, rest, wm, wr, b_sp, b_ab, b_it, b_ms, b_rs):
    blk = _NSL // _RB
    row = pl.BlockSpec((_RB, D), lambda i: (i, 0))
    row_out = pl.BlockSpec((_RB, D), lambda i: (k * blk + i, 0))
    full = pl.BlockSpec((D, D), lambda i: (0, 0))
    vec = pl.BlockSpec((1, D), lambda i: (0, 0))
    return pl.pallas_call(
        _finish_body,
        grid=(blk,),
        in_specs=[row_out, row, row, row, full, full, vec, vec, vec, vec, vec],
        out_specs=row_out,
        out_shape=jax.ShapeDtypeStruct((N, D), jnp.float32),
        input_output_aliases={0: 0},
    )(out_prev, acc, ms, rest, wm, wr, b_sp, b_ab, b_it, b_ms, b_rs)


def kernel(entities, species_tab, species_lw, species_lb, item_tab, item_lw,
           item_lb, ability_tab, ability_lw, ability_lb, moves_tab, moves_lw,
           moves_lb, rest_W, rest_b, mW_species, mb_species, mW_ability,
           mb_ability, mW_item, mb_item, mW_moveset, mb_moveset, mW_rest,
           mb_rest):
    r1 = lambda v: v.reshape(1, D)
    tsp, tab, tit, tmv, tlm, thp, t168, t16 = _build_tables(
        species_tab, species_lw, r1(species_lb), item_tab, item_lw,
        r1(item_lb), ability_tab, ability_lw, r1(ability_lb), moves_tab,
        moves_lw, r1(moves_lb), rest_W, r1(rest_b), jnp.asarray(_HP_TAB),
        mW_species, mW_ability, mW_item)

    ent24 = _transpose_ent(jnp.asarray(entities, jnp.int32).reshape(N, _EC))

    parts = [_sc_gather(k * _NSL, ent24, tsp, tab, tit, tmv, tlm, thp,
                        t168, t16) for k in range(_KSL)]
    out = jnp.zeros((N, D), jnp.float32)
    for k, (acc, ms, rest) in enumerate(parts):
        out = _finish(out, k, acc, ms, rest, mW_moveset, mW_rest,
                      r1(mb_species), r1(mb_ability), r1(mb_item),
                      r1(mb_moveset), r1(mb_rest))
    return out.reshape(B, T, D)


# speedup vs baseline: 1.2099x; 1.0027x over previous
import jax
import jax.numpy as jnp
import numpy as np
from jax import lax
from jax.experimental import pallas as pl
from jax.experimental.pallas import tpu as pltpu
from jax.experimental.pallas import tpu_sc as plsc

B, T = 4096, 12
N = B * T
D = 256
NUM_STATUS = 7

_NC, _NS = 2, 16
_NW = _NC * _NS
_KSL = 4
_NSL = N // _KSL
_ROWS_PER_TILE = _NSL // _NW
_SLAB = 128
_E = 32
_SUB = _SLAB // _E
_SLABS = _ROWS_PER_TILE // _SLAB


def _hp_multihot_np(num_tokens=1024, n_bins=64):
    values = np.arange(num_tokens)
    arr = (np.arange(n_bins)[None] < np.floor(n_bins * values / num_tokens)[:, None]).astype(float)
    extra = values % (num_tokens / n_bins) / (num_tokens / n_bins)
    extra = 2 * extra - 1
    extra_mask = (np.arange(n_bins)[None] <= np.floor(n_bins * values / num_tokens)[:, None]) - arr
    arr = arr + extra_mask * extra[:, None]
    arr = np.where(arr <= 0, -1.0, arr)
    return np.asarray(arr, dtype=np.float32)


_HP_TAB = _hp_multihot_np()


def _ln(x):
    mu = jnp.mean(x, axis=-1, keepdims=True)
    var = jnp.var(x, axis=-1, keepdims=True)
    return (x - mu) * lax.rsqrt(var + 1e-5)


def _dot(a, b):
    return jnp.dot(a, b, preferred_element_type=jnp.float32)


def _tables_body(sp_tab, sp_lw, sp_lb, it_tab, it_lw, it_lb, ab_tab, ab_lw,
                 ab_lb, mv_tab, mv_lw, mv_lb, rest_w, rest_b, hp_tab,
                 mw_sp, mw_ab, mw_it,
                 o_sp, o_ab, o_it, o_mv, o_lm, o_hp, o_168, o_16):
    o_sp[...] = _dot(_ln(_dot(sp_tab[...], sp_lw[...]) + sp_lb[...]), mw_sp[...])
    o_ab[...] = _dot(_ln(_dot(ab_tab[...], ab_lw[...]) + ab_lb[...]), mw_ab[...])
    o_it[...] = _dot(_ln(_dot(it_tab[...], it_lw[...]) + it_lb[...]), mw_it[...])
    tmv = _dot(mv_tab[...], mv_lw[...]) + mv_lb[...]
    o_mv[...] = tmv
    w = rest_w[...]
    o_lm[...] = _dot(tmv, w[0:256])
    o_hp[...] = _dot(hp_tab[...], w[256:320])

    def onehot(rows, vals, div, mod):
        r = lax.broadcasted_iota(jnp.int32, (rows, vals), 0)
        c = lax.broadcasted_iota(jnp.int32, (rows, vals), 1)
        return jnp.where((r // div) % mod == c, 1.0, 0.0).astype(jnp.float32)

    o_168[...] = (_dot(onehot(168, NUM_STATUS, 24, NUM_STATUS), w[324:331]) +
                  _dot(onehot(168, 4, 6, 4), w[335:339]) +
                  _dot(onehot(168, 6, 1, 6), w[339:345]))
    o_16[...] = (_dot(onehot(16, 2, 8, 2), w[320:322]) +
                 _dot(onehot(16, 2, 4, 2), w[322:324]) +
                 _dot(onehot(16, 2, 2, 2), w[333:335]) +
                 _dot(onehot(16, 2, 1, 2), w[331:333]) + rest_b[...])


def _build_tables(sp_tab, sp_lw, sp_lb, it_tab, it_lw, it_lb, ab_tab, ab_lw,
                  ab_lb, mv_tab, mv_lw, mv_lb, rest_w, rest_b, hp_tab,
                  mw_sp, mw_ab, mw_it):
    outs = [jax.ShapeDtypeStruct((r, D), jnp.float32)
            for r in (450, 200, 300, 400, 400, 1024, 168, 16)]
    return pl.pallas_call(_tables_body, out_shape=outs)(
        sp_tab, sp_lw, sp_lb, it_tab, it_lw, it_lb, ab_tab, ab_lw, ab_lb,
        mv_tab, mv_lw, mv_lb, rest_w, rest_b, hp_tab, mw_sp, mw_ab, mw_it)


_EC = 24


def _sc_body(koff, ent, tsp, tab, tit, tmv, tlm, thp, t168, t16,
             acc_out, ms_out, rest_out,
             ent_v, cidx, bufs, s0, s1):
    wid = lax.axis_index("s") * _NC + lax.axis_index("c")

    def slab(si, carry):
        sbase = wid * _ROWS_PER_TILE + si * _SLAB
        pltpu.sync_copy(ent.at[:, pl.ds(koff + sbase, _SLAB)], ent_v)
        for i in range(_SLAB // 16):
            sl = pl.ds(i * 16, 16)
            st = ent_v[6, sl]
            slp = jnp.minimum(jnp.maximum(ent_v[10, sl], 0), 3)
            tx = jnp.minimum(jnp.maximum(ent_v[11, sl], 0), 5)
            cidx[0, sl] = st * 24 + slp * 6 + tx
            cidx[1, sl] = (ent_v[4, sl] * 8 + ent_v[5, sl] * 4 +
                           ent_v[8, sl] * 2 + ent_v[9, sl])

        def chunk(sub, carry2):
            base = sbase + sub * _E
            ix = lambda row: ent_v.at[row, pl.ds(sub * _E, _E)]
            gathers = [
                pltpu.async_copy(tsp.at[ix(0)], bufs[0], s0),
                pltpu.async_copy(tab.at[ix(2)], bufs[1], s0),
                pltpu.async_copy(tit.at[ix(1)], bufs[2], s0),
                pltpu.async_copy(tmv.at[ix(20)], bufs[3], s0),
                pltpu.async_copy(tmv.at[ix(21)], bufs[4], s0),
                pltpu.async_copy(tmv.at[ix(22)], bufs[5], s0),
                pltpu.async_copy(tmv.at[ix(23)], bufs[6], s0),
                pltpu.async_copy(tlm.at[ix(7)], bufs[7], s0),
                pltpu.async_copy(thp.at[ix(3)], bufs[8], s0),
                pltpu.async_copy(t168.at[cidx.at[0, pl.ds(sub * _E, _E)]], bufs[9], s0),
                pltpu.async_copy(t16.at[cidx.at[1, pl.ds(sub * _E, _E)]], bufs[10], s0),
            ]
            for c in gathers:
                c.wait()

            @pl.when(si * _SUB + sub > 0)
            def _drain():
                for b in (bufs[11], bufs[12], bufs[13]):
                    pltpu.make_async_copy(b, acc_out.at[pl.ds(base, _E)], s1).wait()

            def sum_grp(g, carry3):
                sl = pl.ds(pl.multiple_of(g * 32, 32), 32)
                for r in range(_E):
                    bufs[11][r, sl] = bufs[0][r, sl] + bufs[1][r, sl] + bufs[2][r, sl]
                    bufs[12][r, sl] = (bufs[3][r, sl] + bufs[4][r, sl] +
                                       bufs[5][r, sl] + bufs[6][r, sl])
                    bufs[13][r, sl] = (bufs[7][r, sl] + bufs[8][r, sl] +
                                       bufs[9][r, sl] + bufs[10][r, sl])
                return carry3

            lax.fori_loop(0, D // 32, sum_grp, 0)
            pltpu.async_copy(bufs[11], acc_out.at[pl.ds(base, _E)], s1)
            pltpu.async_copy(bufs[12], ms_out.at[pl.ds(base, _E)], s1)
            pltpu.async_copy(bufs[13], rest_out.at[pl.ds(base, _E)], s1)
            return carry2

        return lax.fori_loop(0, _SUB, chunk, carry)

    lax.fori_loop(0, _SLABS, slab, 0)
    for b in (bufs[11], bufs[12], bufs[13]):
        pltpu.make_async_copy(b, acc_out.at[pl.ds(0, _E)], s1).wait()


def _sc_gather(koff, ent24, tsp, tab, tit, tmv, tlm, thp, t168, t16):
    mesh = plsc.VectorSubcoreMesh(core_axis_name="c", subcore_axis_name="s")

    def entry(ent, tsp, tab, tit, tmv, tlm, thp, t168, t16,
              acc_out, ms_out, rest_out, ent_v, cidx,
              b0, b1, b2, b3, b4, b5, b6, b7, b8, b9, b10, b11, b12, b13,
              s0, s1):
        _sc_body(koff, ent, tsp, tab, tit, tmv, tlm, thp, t168, t16,
                 acc_out, ms_out, rest_out, ent_v, cidx,
                 [b0, b1, b2, b3, b4, b5, b6, b7, b8, b9, b10, b11, b12, b13],
                 s0, s1)

    f = pl.kernel(
        entry, mesh=mesh,
        out_type=[jax.ShapeDtypeStruct((_NSL, D), jnp.float32)] * 3,
        scratch_types=(
            [pltpu.VMEM((_EC, _SLAB), jnp.int32),
             pltpu.VMEM((2, _SLAB), jnp.int32)] +
            [pltpu.VMEM((_E, D), jnp.float32)] * 14 +
            [pltpu.SemaphoreType.DMA, pltpu.SemaphoreType.DMA]
        ))
    return f(ent24, tsp, tab, tit, tmv, tlm, thp, t168, t16)


def _transpose_body(x, out):
    out[...] = x[...].T


def _transpose_ent(ents):
    return pl.pallas_call(
        _transpose_body,
        grid=(N // 2048,),
        in_specs=[pl.BlockSpec((2048, _EC), lambda i: (i, 0))],
        out_specs=pl.BlockSpec((_EC, 2048), lambda i: (0, i)),
        out_shape=jax.ShapeDtypeStruct((_EC, N), jnp.int32),
    )(ents)


_RB = 1024


def _finish_body(prev, acc, ms, rest, wm, wr, b_sp, b_ab, b_it, b_ms, b_rs,
                 out):
    del prev
    bias = b_sp[...] + b_ab[...] + b_it[...] + b_ms[...] + b_rs[...]
    b16 = lambda x: x.astype(jnp.bfloat16)
    out[...] = (acc[...] + bias +
                _dot(b16(_ln(ms[...] * 0.25)), b16(wm[...])) +
                _dot(b16(_ln(rest[...])), b16(wr[...])))


def _finish(out_prev, k, acc, ms, rest, wm, wr, b_sp, b_ab, b_it, b_ms, b_rs):
    blk = _NSL // _RB
    row = pl.BlockSpec((_RB, D), lambda i: (i, 0))
    row_out = pl.BlockSpec((_RB, D), lambda i: (k * blk + i, 0))
    full = pl.BlockSpec((D, D), lambda i: (0, 0))
    vec = pl.BlockSpec((1, D), lambda i: (0, 0))
    return pl.pallas_call(
        _finish_body,
        grid=(blk,),
        in_specs=[row_out, row, row, row, full, full, vec, vec, vec, vec, vec],
        out_specs=row_out,
        out_shape=jax.ShapeDtypeStruct((N, D), jnp.float32),
        input_output_aliases={0: 0},
    )(out_prev, acc, ms, rest, wm, wr, b_sp, b_ab, b_it, b_ms, b_rs)


def kernel(entities, species_tab, species_lw, species_lb, item_tab, item_lw,
           item_lb, ability_tab, ability_lw, ability_lb, moves_tab, moves_lw,
           moves_lb, rest_W, rest_b, mW_species, mb_species, mW_ability,
           mb_ability, mW_item, mb_item, mW_moveset, mb_moveset, mW_rest,
           mb_rest):
    r1 = lambda v: v.reshape(1, D)
    tsp, tab, tit, tmv, tlm, thp, t168, t16 = _build_tables(
        species_tab, species_lw, r1(species_lb), item_tab, item_lw,
        r1(item_lb), ability_tab, ability_lw, r1(ability_lb), moves_tab,
        moves_lw, r1(moves_lb), rest_W, r1(rest_b), jnp.asarray(_HP_TAB),
        mW_species, mW_ability, mW_item)

    ent24 = _transpose_ent(jnp.asarray(entities, jnp.int32).reshape(N, _EC))

    parts = [_sc_gather(k * _NSL, ent24, tsp, tab, tit, tmv, tlm, thp,
                        t168, t16) for k in range(_KSL)]
    out = jnp.zeros((N, D), jnp.float32)
    for k, (acc, ms, rest) in enumerate(parts):
        out = _finish(out, k, acc, ms, rest, mW_moveset, mW_rest,
                      r1(mb_species), r1(mb_ability), r1(mb_item),
                      r1(mb_moveset), r1(mb_rest))
    return out.reshape(B, T, D)
